# baseline (speedup 1.0000x reference)
import functools

import jax
import jax.numpy as jnp
from jax import lax
from jax.experimental import pallas as pl
from jax.experimental.pallas import tpu as pltpu
from jax.experimental.pallas import tpu_sc as plsc

N = 10000
E = 320000
NPAD = 10240
F1 = 128
F2 = 64
NC = 2
NS = 16
NW = NC * NS
EW = E // NW
CH = 128
EWP = 10240
NCH = EWP // CH
NGC = 16
NG = NCH // NGC
NR = NPAD // NS


def _seg_sum_body(D, with_cnt, *refs):
    if with_cnt:
        (src3, dst3, x, zrows, zn, ones_c, s_out, cnt_out,
         src_v, dst_v, rows_v, ones_v, acc_sh, cnt_sh,
         lsem, gsem, ssem) = refs
    else:
        (src3, dst3, x, zrows, s_out,
         src_v, dst_v, rows_v, acc_sh, lsem, gsem, ssem) = refs
    cid = lax.axis_index("c")
    sid = lax.axis_index("s")
    wid = cid * NS + sid

    pltpu.sync_copy(zrows.at[pl.ds(sid * NR, NR)],
                    acc_sh.at[pl.ds(sid * NR, NR)])
    if with_cnt:
        pltpu.sync_copy(zn.at[pl.ds(sid * NR, NR)],
                        cnt_sh.at[pl.ds(sid * NR, NR)])
        pltpu.sync_copy(ones_c, ones_v)

    def fire_slab(g, sb):
        pltpu.async_copy(src3.at[wid, pl.ds(g * NGC, NGC)], src_v.at[sb],
                         lsem.at[sb])
        pltpu.async_copy(dst3.at[wid, pl.ds(g * NGC, NGC)], dst_v.at[sb],
                         lsem.at[sb])

    def wait_slab(sb):
        pltpu.make_async_copy(src3.at[wid, pl.ds(0, NGC)], src_v.at[sb],
                              lsem.at[sb]).wait()
        pltpu.make_async_copy(dst3.at[wid, pl.ds(0, NGC)], dst_v.at[sb],
                              lsem.at[sb]).wait()

    def fire_gather(sb, j, b):
        pltpu.async_copy(x.at[src_v.at[sb, j]], rows_v.at[b], gsem.at[b])

    def wait_gather(b):
        pltpu.make_async_copy(x.at[src_v.at[0, 0]], rows_v.at[b],
                              gsem.at[b]).wait()

    def fire_scatter(sb, j, b):
        pltpu.async_copy(rows_v.at[b], acc_sh.at[dst_v.at[sb, j]],
                         ssem.at[b], add=True)

    def wait_scatter(b):
        pltpu.make_async_copy(rows_v.at[b], acc_sh.at[dst_v.at[0, 0]],
                              ssem.at[b]).wait()

    def cnt_add(sb, j):
        if with_cnt:
            pltpu.sync_copy(ones_v, cnt_sh.at[dst_v.at[sb, j]], add=True)

    fire_slab(0, 0)
    plsc.subcore_barrier()

    for g in range(NG):
        sb = g % 2
        wait_slab(sb)
        if g + 1 < NG:
            fire_slab(g + 1, 1 - sb)
        def scatter_sync(j, b, sb=sb):
            fire_scatter(sb, j, b)
            wait_scatter(b)

        fire_gather(sb, 0, 0)

        def pair(t, carry, sb=sb):
            fire_gather(sb, 2 * t + 1, 1)
            wait_gather(0)
            scatter_sync(2 * t, 0)
            cnt_add(sb, 2 * t)
            fire_gather(sb, 2 * t + 2, 0)
            wait_gather(1)
            scatter_sync(2 * t + 1, 1)
            cnt_add(sb, 2 * t + 1)
            return carry

        lax.fori_loop(0, NGC // 2 - 1, pair, 0)
        fire_gather(sb, NGC - 1, 1)
        wait_gather(0)
        scatter_sync(NGC - 2, 0)
        cnt_add(sb, NGC - 2)
        wait_gather(1)
        scatter_sync(NGC - 1, 1)
        cnt_add(sb, NGC - 1)
    plsc.subcore_barrier()

    pltpu.sync_copy(acc_sh.at[pl.ds(sid * NR, NR)],
                    s_out.at[cid, pl.ds(sid * NR, NR)])
    if with_cnt:
        pltpu.sync_copy(cnt_sh.at[pl.ds(sid * NR, NR)],
                        cnt_out.at[cid, pl.ds(sid * NR, NR)])


def _make_seg_sum(D, with_cnt):
    mesh = plsc.VectorSubcoreMesh(core_axis_name="c", subcore_axis_name="s",
                                  num_cores=NC, num_subcores=NS)
    out_type = [jax.ShapeDtypeStruct((NC, NPAD, D), jnp.float32)]
    scratch = [
        pltpu.VMEM((2, NGC, CH), jnp.int32),
        pltpu.VMEM((2, NGC, CH), jnp.int32),
        pltpu.VMEM((2, CH, D), jnp.float32),
    ]
    if with_cnt:
        out_type.append(jax.ShapeDtypeStruct((NC, NPAD), jnp.float32))
        scratch.append(pltpu.VMEM((CH,), jnp.float32))
    scratch.append(pltpu.VMEM_SHARED((NPAD, D), jnp.float32))
    if with_cnt:
        scratch.append(pltpu.VMEM_SHARED((NPAD,), jnp.float32))
    scratch.append(pltpu.SemaphoreType.DMA((2,)))
    scratch.append(pltpu.SemaphoreType.DMA((2,)))
    scratch.append(pltpu.SemaphoreType.DMA((2,)))
    return pl.kernel(functools.partial(_seg_sum_body, D, with_cnt),
                     out_type=out_type, mesh=mesh, scratch_types=scratch)


def _layer1_body(s_ref, c_ref, x_ref, a1_ref, b1_ref, r1_ref, h_ref):
    s = s_ref[0] + s_ref[1]
    cnt = c_ref[0] + c_ref[1]
    mean = s / jnp.maximum(cnt, 1.0)
    h = (jnp.dot(mean, a1_ref[...], preferred_element_type=jnp.float32)
         + b1_ref[...]
         + jnp.dot(x_ref[...], r1_ref[...], preferred_element_type=jnp.float32))
    h_ref[...] = h


def _layer2_body(s_ref, c_ref, h_ref, c2_ref, r2_ref, b2_ref, o_ref):
    s = s_ref[0] + s_ref[1]
    cnt = c_ref[0] + c_ref[1]
    mean = s / jnp.maximum(cnt, 1.0)
    o_ref[...] = (jnp.dot(mean, c2_ref[...], preferred_element_type=jnp.float32)
                  + b2_ref[...]
                  + jnp.dot(h_ref[...], r2_ref[...],
                            preferred_element_type=jnp.float32))


BR = 1000
GRID = N // BR
NCL = 40


def _tc_layer1(s1, cntr, x, a1, b1, r1):
    return pl.pallas_call(
        _layer1_body,
        grid=(GRID,),
        in_specs=[
            pl.BlockSpec((NC, BR, F1), lambda i: (0, i, 0)),
            pl.BlockSpec((NC, BR, 1), lambda i: (0, i, 0)),
            pl.BlockSpec((BR, F1), lambda i: (i, 0)),
            pl.BlockSpec((F1, F1), lambda i: (0, 0)),
            pl.BlockSpec((1, F1), lambda i: (0, 0)),
            pl.BlockSpec((F1, F1), lambda i: (0, 0)),
        ],
        out_specs=pl.BlockSpec((BR, F1), lambda i: (i, 0)),
        out_shape=jax.ShapeDtypeStruct((N, F1), jnp.float32),
    )(s1, cntr, x, a1, b1, r1)


def _tc_layer2(s2, cntr, h, c2, r2, b2):
    return pl.pallas_call(
        _layer2_body,
        grid=(GRID,),
        in_specs=[
            pl.BlockSpec((NC, BR, F1), lambda i: (0, i, 0)),
            pl.BlockSpec((NC, BR, 1), lambda i: (0, i, 0)),
            pl.BlockSpec((BR, F1), lambda i: (i, 0)),
            pl.BlockSpec((F1, NCL), lambda i: (0, 0)),
            pl.BlockSpec((F1, NCL), lambda i: (0, 0)),
            pl.BlockSpec((1, NCL), lambda i: (0, 0)),
        ],
        out_specs=pl.BlockSpec((BR, NCL), lambda i: (i, 0)),
        out_shape=jax.ShapeDtypeStruct((N, NCL), jnp.float32),
    )(s2, cntr, h, c2, r2, b2)


def kernel(g, embeds, W_l1, b_l1, W_r1, W_l2, b_l2, W_r2):
    pad_rows = jnp.broadcast_to(N + jnp.arange(EWP - EW, dtype=jnp.int32),
                                (NW, EWP - EW))
    src2 = jnp.concatenate(
        [g[0].reshape(NW, EW), jnp.zeros((NW, EWP - EW), jnp.int32)],
        axis=1).reshape(NW, NCH, CH)
    dst2 = jnp.concatenate(
        [g[1].reshape(NW, EW), pad_rows], axis=1).reshape(NW, NCH, CH)
    zrows1 = jnp.zeros((NPAD, F1), jnp.float32)
    zn = jnp.zeros((NPAD,), jnp.float32)
    ones_c = jnp.ones((CH,), jnp.float32)

    s1, cnt = _make_seg_sum(F1, True)(src2, dst2, embeds, zrows1, zn, ones_c)
    cntr = cnt.reshape(NC, NPAD, 1)

    a1 = W_l1.T
    r1 = W_r1.T
    b1r = b_l1.reshape(1, F1)
    h = _tc_layer1(s1, cntr, embeds, a1, b1r, r1)

    (s2,) = _make_seg_sum(F1, False)(src2, dst2, h, zrows1)

    return _tc_layer2(s2, cntr, h, W_l2.T, W_r2.T, b_l2.reshape(1, NCL))


# speedup vs baseline: 2.8254x; 2.8254x over previous
import functools

import jax
import jax.numpy as jnp
from jax import lax
from jax.experimental import pallas as pl
from jax.experimental.pallas import tpu as pltpu
from jax.experimental.pallas import tpu_sc as plsc

N = 10000
E = 320000
NPAD = 10240
F1 = 128
F2 = 64
NC = 2
NS = 16
NW = NC * NS
EW = E // NW
CH = 128
EWP = 10240
NCH = EWP // CH
NGC = 16
NG = NCH // NGC
NR = NPAD // NS


def _seg_sum_body(D, with_cnt, *refs):
    if with_cnt:
        (src3, dst3, x, zrows, zn, ones_c, s_out, cnt_out,
         src_v, dst_v, rows_v, ones_v, acc_sh, cnt_sh,
         lsem, gsem, ssem) = refs
    else:
        (src3, dst3, x, zrows, s_out,
         src_v, dst_v, rows_v, acc_sh, lsem, gsem, ssem) = refs
    cid = lax.axis_index("c")
    sid = lax.axis_index("s")
    wid = cid * NS + sid

    pltpu.sync_copy(zrows.at[pl.ds(sid * NR, NR)],
                    acc_sh.at[pl.ds(sid * NR, NR)])
    if with_cnt:
        pltpu.sync_copy(zn.at[pl.ds(sid * NR, NR)],
                        cnt_sh.at[pl.ds(sid * NR, NR)])
        pltpu.sync_copy(ones_c, ones_v)

    def fire_slab(g, sb):
        pltpu.async_copy(src3.at[wid, pl.ds(g * NGC, NGC)], src_v.at[sb],
                         lsem.at[sb])
        pltpu.async_copy(dst3.at[wid, pl.ds(g * NGC, NGC)], dst_v.at[sb],
                         lsem.at[sb])

    def wait_slab(sb):
        pltpu.make_async_copy(src3.at[wid, pl.ds(0, NGC)], src_v.at[sb],
                              lsem.at[sb]).wait()
        pltpu.make_async_copy(dst3.at[wid, pl.ds(0, NGC)], dst_v.at[sb],
                              lsem.at[sb]).wait()

    def fire_gather(sb, j, b):
        pltpu.async_copy(x.at[src_v.at[sb, j]], rows_v.at[b], gsem.at[b])

    def wait_gather(b):
        pltpu.make_async_copy(x.at[src_v.at[0, 0]], rows_v.at[b],
                              gsem.at[b]).wait()

    def fire_scatter(sb, j, b):
        pltpu.async_copy(rows_v.at[b], acc_sh.at[dst_v.at[sb, j]],
                         ssem.at[b], add=True)

    def wait_scatter(b):
        pltpu.make_async_copy(rows_v.at[b], acc_sh.at[dst_v.at[0, 0]],
                              ssem.at[b]).wait()

    def cnt_add(sb, j):
        if with_cnt:
            pltpu.sync_copy(ones_v, cnt_sh.at[dst_v.at[sb, j]], add=True)

    fire_slab(0, 0)
    plsc.subcore_barrier()

    for g in range(NG):
        sb = g % 2
        wait_slab(sb)
        if g + 1 < NG:
            fire_slab(g + 1, 1 - sb)
        def scatter_sync(j, b, sb=sb):
            fire_scatter(sb, j, b)
            wait_scatter(b)

        fire_gather(sb, 0, 0)

        def pair(t, carry, sb=sb):
            fire_gather(sb, 2 * t + 1, 1)
            wait_gather(0)
            scatter_sync(2 * t, 0)
            cnt_add(sb, 2 * t)
            fire_gather(sb, 2 * t + 2, 0)
            wait_gather(1)
            scatter_sync(2 * t + 1, 1)
            cnt_add(sb, 2 * t + 1)
            return carry

        lax.fori_loop(0, NGC // 2 - 1, pair, 0)
        fire_gather(sb, NGC - 1, 1)
        wait_gather(0)
        scatter_sync(NGC - 2, 0)
        cnt_add(sb, NGC - 2)
        wait_gather(1)
        scatter_sync(NGC - 1, 1)
        cnt_add(sb, NGC - 1)
    plsc.subcore_barrier()

    pltpu.sync_copy(acc_sh.at[pl.ds(sid * NR, NR)],
                    s_out.at[cid, pl.ds(sid * NR, NR)])
    if with_cnt:
        pltpu.sync_copy(cnt_sh.at[pl.ds(sid * NR, NR)],
                        cnt_out.at[cid, pl.ds(sid * NR, NR)])


def _make_seg_sum(D, with_cnt):
    mesh = plsc.VectorSubcoreMesh(core_axis_name="c", subcore_axis_name="s",
                                  num_cores=NC, num_subcores=NS)
    out_type = [jax.ShapeDtypeStruct((NC, NPAD, D), jnp.float32)]
    scratch = [
        pltpu.VMEM((2, NGC, CH), jnp.int32),
        pltpu.VMEM((2, NGC, CH), jnp.int32),
        pltpu.VMEM((2, CH, D), jnp.float32),
    ]
    if with_cnt:
        out_type.append(jax.ShapeDtypeStruct((NC, NPAD), jnp.float32))
        scratch.append(pltpu.VMEM((CH,), jnp.float32))
    scratch.append(pltpu.VMEM_SHARED((NPAD, D), jnp.float32))
    if with_cnt:
        scratch.append(pltpu.VMEM_SHARED((NPAD,), jnp.float32))
    scratch.append(pltpu.SemaphoreType.DMA((2,)))
    scratch.append(pltpu.SemaphoreType.DMA((2,)))
    scratch.append(pltpu.SemaphoreType.DMA((2,)))
    return pl.kernel(functools.partial(_seg_sum_body, D, with_cnt),
                     out_type=out_type, mesh=mesh, scratch_types=scratch)


def _layer1_body(s_ref, c_ref, x_ref, a1_ref, b1_ref, r1_ref, h_ref):
    s = s_ref[0] + s_ref[1]
    cnt = c_ref[0] + c_ref[1]
    mean = s / jnp.maximum(cnt, 1.0)
    h = (jnp.dot(mean, a1_ref[...], preferred_element_type=jnp.float32)
         + b1_ref[...]
         + jnp.dot(x_ref[...], r1_ref[...], preferred_element_type=jnp.float32))
    h_ref[...] = h


def _layer2_body(s_ref, c_ref, h_ref, c2_ref, r2_ref, b2_ref, o_ref):
    s = s_ref[0] + s_ref[1]
    cnt = c_ref[0] + c_ref[1]
    mean = s / jnp.maximum(cnt, 1.0)
    o_ref[...] = (jnp.dot(mean, c2_ref[...], preferred_element_type=jnp.float32)
                  + b2_ref[...]
                  + jnp.dot(h_ref[...], r2_ref[...],
                            preferred_element_type=jnp.float32))


BR = 1000
GRID = N // BR
NCL = 40


def _tc_layer1(s1, cntr, x, a1, b1, r1):
    return pl.pallas_call(
        _layer1_body,
        grid=(GRID,),
        in_specs=[
            pl.BlockSpec((NC, BR, F1), lambda i: (0, i, 0)),
            pl.BlockSpec((NC, BR, 1), lambda i: (0, i, 0)),
            pl.BlockSpec((BR, F1), lambda i: (i, 0)),
            pl.BlockSpec((F1, F1), lambda i: (0, 0)),
            pl.BlockSpec((1, F1), lambda i: (0, 0)),
            pl.BlockSpec((F1, F1), lambda i: (0, 0)),
        ],
        out_specs=pl.BlockSpec((BR, F1), lambda i: (i, 0)),
        out_shape=jax.ShapeDtypeStruct((N, F1), jnp.float32),
    )(s1, cntr, x, a1, b1, r1)


def _tc_layer2(s2, cntr, h, c2, r2, b2):
    return pl.pallas_call(
        _layer2_body,
        grid=(GRID,),
        in_specs=[
            pl.BlockSpec((NC, BR, F1), lambda i: (0, i, 0)),
            pl.BlockSpec((NC, BR, 1), lambda i: (0, i, 0)),
            pl.BlockSpec((BR, F1), lambda i: (i, 0)),
            pl.BlockSpec((F1, NCL), lambda i: (0, 0)),
            pl.BlockSpec((F1, NCL), lambda i: (0, 0)),
            pl.BlockSpec((1, NCL), lambda i: (0, 0)),
        ],
        out_specs=pl.BlockSpec((BR, NCL), lambda i: (i, 0)),
        out_shape=jax.ShapeDtypeStruct((N, NCL), jnp.float32),
    )(s2, cntr, h, c2, r2, b2)


def kernel(g, embeds, W_l1, b_l1, W_r1, W_l2, b_l2, W_r2):
    pad_rows = jnp.broadcast_to(N + jnp.arange(EWP - EW, dtype=jnp.int32),
                                (NW, EWP - EW))
    src_pads = jnp.broadcast_to(jnp.arange(EWP - EW, dtype=jnp.int32),
                                (NW, EWP - EW))
    src2 = jnp.concatenate(
        [g[0].reshape(NW, EW), src_pads], axis=1).reshape(NW, NCH, CH)
    dst2 = jnp.concatenate(
        [g[1].reshape(NW, EW), pad_rows], axis=1).reshape(NW, NCH, CH)
    zrows1 = jnp.zeros((NPAD, F1), jnp.float32)
    zn = jnp.zeros((NPAD,), jnp.float32)
    ones_c = jnp.ones((CH,), jnp.float32)

    s1, cnt = _make_seg_sum(F1, True)(src2, dst2, embeds, zrows1, zn, ones_c)
    cntr = cnt.reshape(NC, NPAD, 1)

    a1 = W_l1.T
    r1 = W_r1.T
    b1r = b_l1.reshape(1, F1)
    h = _tc_layer1(s1, cntr, embeds, a1, b1r, r1)

    (s2,) = _make_seg_sum(F1, False)(src2, dst2, h, zrows1)

    return _tc_layer2(s2, cntr, h, W_l2.T, W_r2.T, b_l2.reshape(1, NCL))
